# baseline (speedup 1.0000x reference)
import jax
import jax.numpy as jnp
from jax.experimental import pallas as pl
from jax.experimental.pallas import tpu as pltpu

N = 10000
NFEAT = 128
NEMB = 128
BM = 200


def _fused_body(x_ref, w1_ref, w2_ref, adj1_ref, adj2_ref, b1_ref, b2_ref,
                wa_ref, emb1_ref, emb2_ref, emb_ref, s1_ref, s2_ref):
    @pl.when(pl.program_id(0) == 0)
    def _():
        x = x_ref[...]
        s1_ref[...] = jnp.dot(x, w1_ref[...], preferred_element_type=jnp.float32)
        s2_ref[...] = jnp.dot(x, w2_ref[...], preferred_element_type=jnp.float32)

    e1 = jnp.dot(adj1_ref[...], s1_ref[...], precision=jax.lax.Precision.HIGHEST,
                 preferred_element_type=jnp.float32) + b1_ref[...]
    e2 = jnp.dot(adj2_ref[...], s2_ref[...], precision=jax.lax.Precision.HIGHEST,
                 preferred_element_type=jnp.float32) + b2_ref[...]
    emb1_ref[...] = e1
    emb2_ref[...] = e2
    wa = wa_ref[...]
    w1 = jnp.sum(e1 * wa, axis=1, keepdims=True)
    w2 = jnp.sum(e2 * wa, axis=1, keepdims=True)
    m = jnp.maximum(w1, w2)
    p1 = jnp.exp(w1 - m)
    p2 = jnp.exp(w2 - m)
    inv = 1.0 / (p1 + p2)
    emb_ref[...] = (p1 * inv) * e1 + (p2 * inv) * e2


@jax.jit
def kernel(x, adj1, adj2, W1, b1, W2, b2, Wa):
    b1r = b1.reshape(1, NEMB)
    b2r = b2.reshape(1, NEMB)
    war = Wa.reshape(1, NEMB)

    grid = (N // BM,)
    emb1, emb2, emb = pl.pallas_call(
        _fused_body,
        grid=grid,
        in_specs=[
            pl.BlockSpec((N, NFEAT), lambda i: (0, 0)),
            pl.BlockSpec((NFEAT, NEMB), lambda i: (0, 0)),
            pl.BlockSpec((NFEAT, NEMB), lambda i: (0, 0)),
            pl.BlockSpec((BM, N), lambda i: (i, 0)),
            pl.BlockSpec((BM, N), lambda i: (i, 0)),
            pl.BlockSpec((1, NEMB), lambda i: (0, 0)),
            pl.BlockSpec((1, NEMB), lambda i: (0, 0)),
            pl.BlockSpec((1, NEMB), lambda i: (0, 0)),
        ],
        out_specs=[
            pl.BlockSpec((BM, NEMB), lambda i: (i, 0)),
            pl.BlockSpec((BM, NEMB), lambda i: (i, 0)),
            pl.BlockSpec((BM, NEMB), lambda i: (i, 0)),
        ],
        out_shape=(
            jax.ShapeDtypeStruct((N, NEMB), jnp.float32),
            jax.ShapeDtypeStruct((N, NEMB), jnp.float32),
            jax.ShapeDtypeStruct((N, NEMB), jnp.float32),
        ),
        scratch_shapes=[
            pltpu.VMEM((N, NEMB), jnp.float32),
            pltpu.VMEM((N, NEMB), jnp.float32),
        ],
        compiler_params=pltpu.CompilerParams(
            dimension_semantics=("arbitrary",),
        ),
    )(x, W1, W2, adj1, adj2, b1r, b2r, war)
    return (emb1, emb2, emb)


# speedup vs baseline: 2.7826x; 2.7826x over previous
import jax
import jax.numpy as jnp
from jax.experimental import pallas as pl
from jax.experimental.pallas import tpu as pltpu

N = 10000
NFEAT = 128
NEMB = 128
BM = 200


def _fused_body(x_ref, w1_ref, w2_ref, adj1_ref, adj2_ref, b1_ref, b2_ref,
                wa_ref, emb1_ref, emb2_ref, emb_ref, s1_ref, s2_ref):
    @pl.when(pl.program_id(0) == 0)
    def _():
        x = x_ref[...]
        s1_ref[...] = jnp.dot(x, w1_ref[...], preferred_element_type=jnp.float32)
        s2_ref[...] = jnp.dot(x, w2_ref[...], preferred_element_type=jnp.float32)

    e1 = jnp.dot(adj1_ref[...], s1_ref[...],
                 preferred_element_type=jnp.float32) + b1_ref[...]
    e2 = jnp.dot(adj2_ref[...], s2_ref[...],
                 preferred_element_type=jnp.float32) + b2_ref[...]
    emb1_ref[...] = e1
    emb2_ref[...] = e2
    wa = wa_ref[...].astype(jnp.bfloat16).astype(jnp.float32)
    e1b = e1.astype(jnp.bfloat16).astype(jnp.float32)
    e2b = e2.astype(jnp.bfloat16).astype(jnp.float32)
    w1 = jnp.sum(e1b * wa, axis=1, keepdims=True)
    w2 = jnp.sum(e2b * wa, axis=1, keepdims=True)
    m = jnp.maximum(w1, w2)
    p1 = jnp.exp(w1 - m)
    p2 = jnp.exp(w2 - m)
    inv = 1.0 / (p1 + p2)
    emb_ref[...] = (p1 * inv) * e1 + (p2 * inv) * e2


@jax.jit
def kernel(x, adj1, adj2, W1, b1, W2, b2, Wa):
    b1r = b1.reshape(1, NEMB)
    b2r = b2.reshape(1, NEMB)
    war = Wa.reshape(1, NEMB)

    grid = (N // BM,)
    emb1, emb2, emb = pl.pallas_call(
        _fused_body,
        grid=grid,
        in_specs=[
            pl.BlockSpec((N, NFEAT), lambda i: (0, 0)),
            pl.BlockSpec((NFEAT, NEMB), lambda i: (0, 0)),
            pl.BlockSpec((NFEAT, NEMB), lambda i: (0, 0)),
            pl.BlockSpec((BM, N), lambda i: (i, 0)),
            pl.BlockSpec((BM, N), lambda i: (i, 0)),
            pl.BlockSpec((1, NEMB), lambda i: (0, 0)),
            pl.BlockSpec((1, NEMB), lambda i: (0, 0)),
            pl.BlockSpec((1, NEMB), lambda i: (0, 0)),
        ],
        out_specs=[
            pl.BlockSpec((BM, NEMB), lambda i: (i, 0)),
            pl.BlockSpec((BM, NEMB), lambda i: (i, 0)),
            pl.BlockSpec((BM, NEMB), lambda i: (i, 0)),
        ],
        out_shape=(
            jax.ShapeDtypeStruct((N, NEMB), jnp.float32),
            jax.ShapeDtypeStruct((N, NEMB), jnp.float32),
            jax.ShapeDtypeStruct((N, NEMB), jnp.float32),
        ),
        scratch_shapes=[
            pltpu.VMEM((N, NEMB), jnp.float32),
            pltpu.VMEM((N, NEMB), jnp.float32),
        ],
        compiler_params=pltpu.CompilerParams(
            dimension_semantics=("arbitrary",),
        ),
    )(x, W1, W2, adj1, adj2, b1r, b2r, war)
    return (emb1, emb2, emb)
